# baseline (speedup 1.0000x reference)
import functools

import jax
import jax.numpy as jnp
from jax import lax
from jax.experimental import pallas as pl
from jax.experimental.pallas import tpu as pltpu
from jax.experimental.pallas import tpu_sc as plsc

N = 10000
E = 160000
R = 4
G = 16
NW = 32
NP_ = 10240
NSEGP = R * NP_
EP = E + 512

DI = 128
CRT = 256
CPT = 5
K = 128
ACR = CRT + 32
NCH = NW * CPT


def _make_sc_agg():
    mesh = plsc.VectorSubcoreMesh(core_axis_name="c", subcore_axis_name="s")

    @functools.partial(
        pl.kernel,
        mesh=mesh,
        out_type=jax.ShapeDtypeStruct((NSEGP, DI), jnp.float32),
        scratch_types=[
            pltpu.VMEM((K,), jnp.int32),
            pltpu.VMEM((K + 16,), jnp.int32),
            pltpu.VMEM((K, DI), jnp.float32),
            pltpu.VMEM((16,), jnp.int32),
            pltpu.VMEM((16,), jnp.int32),
            pltpu.VMEM((ACR, DI), jnp.float32),
            pltpu.SemaphoreType.DMA,
        ],
    )
    def agg_kernel(x_hbm, src_hbm, seg_hbm, boundsT_hbm, consts_hbm, out_hbm,
                   srcv, segv, rows, bvm, cvm, acc, sem):
        core = lax.axis_index("c")
        sub = lax.axis_index("s")
        w = sub * 2 + core

        lanes = lax.iota(jnp.int32, 16)
        zf16 = jnp.zeros((16,), jnp.float32)

        pltpu.sync_copy(consts_hbm, cvm)
        cv = cvm[pl.ds(0, 16)]
        k_t = cv[0]
        c16_t = cv[1]
        acr_t = cv[2]
        cpt_t = cv[3]

        def chunk_body(t, _):
            j = w * CPT + t
            base = j * CRT

            def zc(r, _):
                def zcc(c, _):
                    acc[r, pl.ds(c * 16, 16)] = zf16
                    return 0
                lax.fori_loop(0, c16_t, zcc, 0)
                return 0
            lax.fori_loop(0, acr_t, zc, 0)

            pltpu.sync_copy(boundsT_hbm.at[pl.ds(j * 16, 16)], bvm)
            bv = bvm[pl.ds(0, 16)]
            e0 = bv[0]
            e1 = bv[1]
            astart = pl.multiple_of((e0 // 8) * 8, 8)
            nb = (e1 - astart + K - 1) // K

            def body(b, _):
                ab = pl.multiple_of(astart + b * K, 8)
                pltpu.sync_copy(src_hbm.at[pl.ds(ab, K)], srcv)
                pltpu.sync_copy(seg_hbm.at[pl.ds(ab, K)], segv.at[pl.ds(0, K)])
                for v in range(K // 16):
                    sl = pl.ds(v * 16, 16)
                    eid = ab + v * 16 + lanes
                    m = (eid >= e0) & (eid < e1)
                    srcv[sl] = jnp.where(m, srcv[sl], 0)
                    segv[sl] = jnp.where(m, segv[sl] - base, CRT)
                pltpu.async_copy(x_hbm.at[srcv], rows, sem).wait()

                def ebody(e, _):
                    row = segv[pl.ds(e, 16)][0]

                    def cbody(c, _):
                        sl = pl.ds(c * 16, 16)
                        acc[row, sl] = acc[row, sl] + rows[e, sl]
                        return 0
                    lax.fori_loop(0, c16_t, cbody, 0)
                    return 0
                lax.fori_loop(0, k_t, ebody, 0)
                return 0
            lax.fori_loop(0, nb, body, 0)

            pltpu.sync_copy(acc.at[pl.ds(0, CRT)],
                            out_hbm.at[pl.ds(base, CRT)])
            return 0
        lax.fori_loop(0, cpt_t, chunk_body, 0)

    def call(x_col, src, seg, boundsT):
        consts = jnp.array([K, DI // 16, ACR, CPT] + [0] * 12, jnp.int32)
        return agg_kernel(x_col, src, seg, boundsT, consts)
    return call


def _tc_layer(agg_cbs, cnt, x, w_cbs, root, b, relu, bn=256):
    CB = len(agg_cbs)
    _, np_, _ = agg_cbs[0].shape
    di = x.shape[1]
    do = w_cbs[0].shape[2]
    grid = (np_ // bn,)

    def body(*refs):
        agg_refs = refs[:CB]
        cnt_ref, x_ref = refs[CB], refs[CB + 1]
        w_refs = refs[CB + 2:2 * CB + 2]
        root_ref, b_ref, o_ref = refs[2 * CB + 2:]
        acc = jnp.dot(x_ref[...], root_ref[...],
                      preferred_element_type=jnp.float32)
        acc += b_ref[...]
        for r in range(R):
            inv = 1.0 / jnp.maximum(cnt_ref[:, r:r + 1], 1.0)
            pr = jnp.zeros_like(acc)
            for cb in range(CB):
                pr += jnp.dot(agg_refs[cb][r], w_refs[cb][r],
                              preferred_element_type=jnp.float32)
            acc += pr * inv
        if relu:
            acc = jnp.maximum(acc, 0.0)
        o_ref[...] = acc

    in_specs = (
        [pl.BlockSpec((R, bn, DI), lambda i: (0, i, 0)) for _ in range(CB)]
        + [pl.BlockSpec((bn, R), lambda i: (i, 0)),
           pl.BlockSpec((bn, di), lambda i: (i, 0))]
        + [pl.BlockSpec((R, DI, do), lambda i: (0, 0, 0)) for _ in range(CB)]
        + [pl.BlockSpec((di, do), lambda i: (0, 0)),
           pl.BlockSpec((1, do), lambda i: (0, 0))]
    )
    return pl.pallas_call(
        body,
        grid=grid,
        in_specs=in_specs,
        out_specs=pl.BlockSpec((bn, do), lambda i: (i, 0)),
        out_shape=jax.ShapeDtypeStruct((np_, do), jnp.float32),
    )(*agg_cbs, cnt, x, *w_cbs, root, b)


def _tc_pool(batch16, h, bn=512):
    np_, do = h.shape
    grid = (np_ // bn,)
    last = np_ // bn - 1

    def body(batch_ref, h_ref, o_ref, sum_ref, cnt_ref):
        i = pl.program_id(0)

        @pl.when(i == 0)
        def _():
            sum_ref[...] = jnp.zeros_like(sum_ref)
            cnt_ref[...] = jnp.zeros_like(cnt_ref)

        gids = lax.broadcasted_iota(jnp.int32, (bn, G), 1)
        onehot = (batch_ref[...] == gids).astype(jnp.float32)
        sum_ref[...] += lax.dot_general(
            onehot, h_ref[...], (((0,), (0,)), ((), ())),
            preferred_element_type=jnp.float32)
        cnt_ref[...] += lax.dot_general(
            onehot, jnp.ones((bn, do), jnp.float32), (((0,), (0,)), ((), ())),
            preferred_element_type=jnp.float32)

        @pl.when(i == last)
        def _():
            o_ref[...] = sum_ref[...] / jnp.maximum(cnt_ref[...], 1.0)

    return pl.pallas_call(
        body,
        grid=grid,
        in_specs=[
            pl.BlockSpec((bn, G), lambda i: (i, 0)),
            pl.BlockSpec((bn, do), lambda i: (i, 0)),
        ],
        out_specs=pl.BlockSpec((G, do), lambda i: (0, 0)),
        out_shape=jax.ShapeDtypeStruct((G, do), jnp.float32),
        scratch_shapes=[
            pltpu.VMEM((G, do), jnp.float32),
            pltpu.VMEM((G, do), jnp.float32),
        ],
    )(batch16, h)


def _sc_layer_agg(sc, h, src_p, seg_p, boundsT):
    CB = h.shape[1] // DI
    cols = jnp.split(h, CB, axis=1)
    return [sc(c, src_p, seg_p, boundsT).reshape(R, NP_, DI) for c in cols]


def kernel(x, edge_index, edge_attr, batch, w1, root1, b1, w2, root2, b2,
           w3, root3, b3, w4, root4, b4):
    x = x.astype(jnp.float32)
    src = edge_index[0].astype(jnp.int32)
    dst = edge_index[1].astype(jnp.int32)
    et = edge_attr.astype(jnp.int32)

    seg = et * NP_ + dst
    seg_s, src_s = lax.sort_key_val(seg, src)
    src_p = jnp.pad(src_s, (0, EP - E))
    seg_p = jnp.pad(seg_s, (0, EP - E), constant_values=NSEGP)

    b_ = jnp.searchsorted(seg_s, jnp.arange(NCH + 1) * CRT)
    b_ = jnp.pad(b_, (0, 16), constant_values=E).astype(jnp.int32)
    win = jnp.arange(NCH)[:, None] + jnp.arange(16)[None, :]
    boundsT = b_[win].reshape(-1)

    x_p = jnp.pad(x, ((0, NP_ - N), (0, 0)))
    batch_p = jnp.pad(batch.astype(jnp.int32), (0, NP_ - N),
                      constant_values=G)
    batch16 = jnp.broadcast_to(batch_p[:, None], (NP_, G))

    OUTP = 128
    w4p = jnp.pad(w4, ((0, 0), (0, 0), (0, OUTP - w4.shape[2])))
    root4p = jnp.pad(root4, ((0, 0), (0, OUTP - root4.shape[1])))
    b4p = jnp.pad(b4, (0, OUTP - b4.shape[0]))

    sc = _make_sc_agg()

    def wsplit(w):
        return [w[:, cb * DI:(cb + 1) * DI, :] for cb in range(w.shape[1] // DI)]

    ones_tab = jnp.ones((NP_, DI), jnp.float32)
    cnt_raw = sc(ones_tab, src_p, seg_p, boundsT)
    cnt = cnt_raw[:, 0].reshape(R, NP_).T

    a1 = _sc_layer_agg(sc, x_p, src_p, seg_p, boundsT)
    h = _tc_layer(a1, cnt, x_p, wsplit(w1), root1, b1.reshape(1, -1),
                  relu=True)

    a2 = _sc_layer_agg(sc, h, src_p, seg_p, boundsT)
    h = _tc_layer(a2, cnt, h, wsplit(w2), root2, b2.reshape(1, -1),
                  relu=True)

    a3 = _sc_layer_agg(sc, h, src_p, seg_p, boundsT)
    h = _tc_layer(a3, cnt, h, wsplit(w3), root3, b3.reshape(1, -1),
                  relu=True)

    a4 = _sc_layer_agg(sc, h, src_p, seg_p, boundsT)
    h = _tc_layer(a4, cnt, h, wsplit(w4p), root4p, b4p.reshape(1, -1),
                  relu=False)

    pooled = _tc_pool(batch16, h)
    return pooled[:, :19]


# speedup vs baseline: 1.0734x; 1.0734x over previous
import functools

import jax
import jax.numpy as jnp
from jax import lax
from jax.experimental import pallas as pl
from jax.experimental.pallas import tpu as pltpu
from jax.experimental.pallas import tpu_sc as plsc

N = 10000
E = 160000
R = 4
G = 16
NW = 32
NP_ = 10240
NSEGP = R * NP_
EP = E + 512

DI = 128
CRT = 256
CPT = 5
K = 128
ACR = CRT + 32
NCH = NW * CPT


def _make_sc_agg():
    mesh = plsc.VectorSubcoreMesh(core_axis_name="c", subcore_axis_name="s")

    @functools.partial(
        pl.kernel,
        mesh=mesh,
        out_type=jax.ShapeDtypeStruct((NSEGP, DI), jnp.float32),
        scratch_types=[
            pltpu.VMEM((K,), jnp.int32),
            pltpu.VMEM((K + 16,), jnp.int32),
            pltpu.VMEM((K, DI), jnp.float32),
            pltpu.VMEM((16,), jnp.int32),
            pltpu.VMEM((16,), jnp.int32),
            pltpu.VMEM((ACR, DI), jnp.float32),
            pltpu.SemaphoreType.DMA,
        ],
    )
    def agg_kernel(x_hbm, src_hbm, seg_hbm, boundsT_hbm, consts_hbm, out_hbm,
                   srcv, segv, rows, bvm, cvm, acc, sem):
        core = lax.axis_index("c")
        sub = lax.axis_index("s")
        w = sub * 2 + core

        lanes = lax.iota(jnp.int32, 16)
        zf16 = jnp.zeros((16,), jnp.float32)

        pltpu.sync_copy(consts_hbm, cvm)
        cv = cvm[pl.ds(0, 16)]
        k_t = cv[0]
        c16_t = cv[1]
        acr_t = cv[2]
        cpt_t = cv[3]

        def chunk_body(t, _):
            j = w * CPT + t
            base = j * CRT

            def zc(r, _):
                for c in range(DI // 16):
                    acc[r, pl.ds(c * 16, 16)] = zf16
                return 0
            lax.fori_loop(0, acr_t, zc, 0)

            pltpu.sync_copy(boundsT_hbm.at[pl.ds(j * 16, 16)], bvm)
            bv = bvm[pl.ds(0, 16)]
            e0 = bv[0]
            e1 = bv[1]
            astart = pl.multiple_of((e0 // 8) * 8, 8)
            nb = (e1 - astart + K - 1) // K

            def body(b, _):
                ab = pl.multiple_of(astart + b * K, 8)
                pltpu.sync_copy(src_hbm.at[pl.ds(ab, K)], srcv)
                pltpu.sync_copy(seg_hbm.at[pl.ds(ab, K)], segv.at[pl.ds(0, K)])
                for v in range(K // 16):
                    sl = pl.ds(v * 16, 16)
                    eid = ab + v * 16 + lanes
                    m = (eid >= e0) & (eid < e1)
                    srcv[sl] = jnp.where(m, srcv[sl], 0)
                    segv[sl] = jnp.where(m, segv[sl] - base, CRT)
                pltpu.async_copy(x_hbm.at[srcv], rows, sem).wait()

                def ebody(e, _):
                    row = segv[pl.ds(e, 16)][0]
                    for c in range(DI // 16):
                        sl = pl.ds(c * 16, 16)
                        acc[row, sl] = acc[row, sl] + rows[e, sl]
                    return 0
                lax.fori_loop(0, k_t, ebody, 0)
                return 0
            lax.fori_loop(0, nb, body, 0)

            pltpu.sync_copy(acc.at[pl.ds(0, CRT)],
                            out_hbm.at[pl.ds(base, CRT)])
            return 0
        lax.fori_loop(0, cpt_t, chunk_body, 0)

    def call(x_col, src, seg, boundsT):
        consts = jnp.array([K, DI // 16, ACR, CPT] + [0] * 12, jnp.int32)
        return agg_kernel(x_col, src, seg, boundsT, consts)
    return call


def _tc_layer(agg_cbs, cnt, x, w_cbs, root, b, relu, bn=256):
    CB = len(agg_cbs)
    _, np_, _ = agg_cbs[0].shape
    di = x.shape[1]
    do = w_cbs[0].shape[2]
    grid = (np_ // bn,)

    def body(*refs):
        agg_refs = refs[:CB]
        cnt_ref, x_ref = refs[CB], refs[CB + 1]
        w_refs = refs[CB + 2:2 * CB + 2]
        root_ref, b_ref, o_ref = refs[2 * CB + 2:]
        acc = jnp.dot(x_ref[...], root_ref[...],
                      preferred_element_type=jnp.float32)
        acc += b_ref[...]
        for r in range(R):
            inv = 1.0 / jnp.maximum(cnt_ref[:, r:r + 1], 1.0)
            pr = jnp.zeros_like(acc)
            for cb in range(CB):
                pr += jnp.dot(agg_refs[cb][r], w_refs[cb][r],
                              preferred_element_type=jnp.float32)
            acc += pr * inv
        if relu:
            acc = jnp.maximum(acc, 0.0)
        o_ref[...] = acc

    in_specs = (
        [pl.BlockSpec((R, bn, DI), lambda i: (0, i, 0)) for _ in range(CB)]
        + [pl.BlockSpec((bn, R), lambda i: (i, 0)),
           pl.BlockSpec((bn, di), lambda i: (i, 0))]
        + [pl.BlockSpec((R, DI, do), lambda i: (0, 0, 0)) for _ in range(CB)]
        + [pl.BlockSpec((di, do), lambda i: (0, 0)),
           pl.BlockSpec((1, do), lambda i: (0, 0))]
    )
    return pl.pallas_call(
        body,
        grid=grid,
        in_specs=in_specs,
        out_specs=pl.BlockSpec((bn, do), lambda i: (i, 0)),
        out_shape=jax.ShapeDtypeStruct((np_, do), jnp.float32),
    )(*agg_cbs, cnt, x, *w_cbs, root, b)


def _tc_pool(batch16, h, bn=512):
    np_, do = h.shape
    grid = (np_ // bn,)
    last = np_ // bn - 1

    def body(batch_ref, h_ref, o_ref, sum_ref, cnt_ref):
        i = pl.program_id(0)

        @pl.when(i == 0)
        def _():
            sum_ref[...] = jnp.zeros_like(sum_ref)
            cnt_ref[...] = jnp.zeros_like(cnt_ref)

        gids = lax.broadcasted_iota(jnp.int32, (bn, G), 1)
        onehot = (batch_ref[...] == gids).astype(jnp.float32)
        sum_ref[...] += lax.dot_general(
            onehot, h_ref[...], (((0,), (0,)), ((), ())),
            preferred_element_type=jnp.float32)
        cnt_ref[...] += lax.dot_general(
            onehot, jnp.ones((bn, do), jnp.float32), (((0,), (0,)), ((), ())),
            preferred_element_type=jnp.float32)

        @pl.when(i == last)
        def _():
            o_ref[...] = sum_ref[...] / jnp.maximum(cnt_ref[...], 1.0)

    return pl.pallas_call(
        body,
        grid=grid,
        in_specs=[
            pl.BlockSpec((bn, G), lambda i: (i, 0)),
            pl.BlockSpec((bn, do), lambda i: (i, 0)),
        ],
        out_specs=pl.BlockSpec((G, do), lambda i: (0, 0)),
        out_shape=jax.ShapeDtypeStruct((G, do), jnp.float32),
        scratch_shapes=[
            pltpu.VMEM((G, do), jnp.float32),
            pltpu.VMEM((G, do), jnp.float32),
        ],
    )(batch16, h)


def _sc_layer_agg(sc, h, src_p, seg_p, boundsT):
    CB = h.shape[1] // DI
    cols = jnp.split(h, CB, axis=1)
    return [sc(c, src_p, seg_p, boundsT).reshape(R, NP_, DI) for c in cols]


def kernel(x, edge_index, edge_attr, batch, w1, root1, b1, w2, root2, b2,
           w3, root3, b3, w4, root4, b4):
    x = x.astype(jnp.float32)
    src = edge_index[0].astype(jnp.int32)
    dst = edge_index[1].astype(jnp.int32)
    et = edge_attr.astype(jnp.int32)

    seg = et * NP_ + dst
    seg_s, src_s = lax.sort_key_val(seg, src)
    src_p = jnp.pad(src_s, (0, EP - E))
    seg_p = jnp.pad(seg_s, (0, EP - E), constant_values=NSEGP)

    b_ = jnp.searchsorted(seg_s, jnp.arange(NCH + 1) * CRT)
    b_ = jnp.pad(b_, (0, 16), constant_values=E).astype(jnp.int32)
    win = jnp.arange(NCH)[:, None] + jnp.arange(16)[None, :]
    boundsT = b_[win].reshape(-1)

    x_p = jnp.pad(x, ((0, NP_ - N), (0, 0)))
    batch_p = jnp.pad(batch.astype(jnp.int32), (0, NP_ - N),
                      constant_values=G)
    batch16 = jnp.broadcast_to(batch_p[:, None], (NP_, G))

    OUTP = 128
    w4p = jnp.pad(w4, ((0, 0), (0, 0), (0, OUTP - w4.shape[2])))
    root4p = jnp.pad(root4, ((0, 0), (0, OUTP - root4.shape[1])))
    b4p = jnp.pad(b4, (0, OUTP - b4.shape[0]))

    sc = _make_sc_agg()

    def wsplit(w):
        return [w[:, cb * DI:(cb + 1) * DI, :] for cb in range(w.shape[1] // DI)]

    ones_tab = jnp.ones((NP_, DI), jnp.float32)
    cnt_raw = sc(ones_tab, src_p, seg_p, boundsT)
    cnt = cnt_raw[:, 0].reshape(R, NP_).T

    a1 = _sc_layer_agg(sc, x_p, src_p, seg_p, boundsT)
    h = _tc_layer(a1, cnt, x_p, wsplit(w1), root1, b1.reshape(1, -1),
                  relu=True)

    a2 = _sc_layer_agg(sc, h, src_p, seg_p, boundsT)
    h = _tc_layer(a2, cnt, h, wsplit(w2), root2, b2.reshape(1, -1),
                  relu=True)

    a3 = _sc_layer_agg(sc, h, src_p, seg_p, boundsT)
    h = _tc_layer(a3, cnt, h, wsplit(w3), root3, b3.reshape(1, -1),
                  relu=True)

    a4 = _sc_layer_agg(sc, h, src_p, seg_p, boundsT)
    h = _tc_layer(a4, cnt, h, wsplit(w4p), root4p, b4p.reshape(1, -1),
                  relu=False)

    pooled = _tc_pool(batch16, h)
    return pooled[:, :19]


# speedup vs baseline: 1.0878x; 1.0134x over previous
import functools

import jax
import jax.numpy as jnp
from jax import lax
from jax.experimental import pallas as pl
from jax.experimental.pallas import tpu as pltpu
from jax.experimental.pallas import tpu_sc as plsc

N = 10000
E = 160000
R = 4
G = 16
NW = 32
NP_ = 10240
NSEGP = R * NP_
EP = E + 512

DI = 128
CRT = 256
CPT = 5
K = 128
ACR = CRT + 32
NCH = NW * CPT


def _make_sc_agg():
    mesh = plsc.VectorSubcoreMesh(core_axis_name="c", subcore_axis_name="s")

    @functools.partial(
        pl.kernel,
        mesh=mesh,
        out_type=jax.ShapeDtypeStruct((NSEGP, DI), jnp.float32),
        scratch_types=[
            pltpu.VMEM((K,), jnp.int32),
            pltpu.VMEM((K + 16,), jnp.int32),
            pltpu.VMEM((K, DI), jnp.float32),
            pltpu.VMEM((16,), jnp.int32),
            pltpu.VMEM((16,), jnp.int32),
            pltpu.VMEM((ACR, DI), jnp.float32),
            pltpu.SemaphoreType.DMA,
        ],
    )
    def agg_kernel(x_hbm, src_hbm, seg_hbm, boundsT_hbm, consts_hbm, out_hbm,
                   srcv, segv, rows, bvm, cvm, acc, sem):
        core = lax.axis_index("c")
        sub = lax.axis_index("s")
        w = sub * 2 + core

        lanes = lax.iota(jnp.int32, 16)
        zf16 = jnp.zeros((16,), jnp.float32)

        pltpu.sync_copy(consts_hbm, cvm)
        cv = cvm[pl.ds(0, 16)]
        k_t = cv[0]
        c16_t = cv[1]
        acr_t = cv[2]
        cpt_t = cv[3]
        k2_t = cv[4]

        def chunk_body(t, _):
            j = w * CPT + t
            base = j * CRT

            def zc(r, _):
                for c in range(DI // 16):
                    acc[r, pl.ds(c * 16, 16)] = zf16
                return 0
            lax.fori_loop(0, acr_t, zc, 0)

            pltpu.sync_copy(boundsT_hbm.at[pl.ds(j * 16, 16)], bvm)
            bv = bvm[pl.ds(0, 16)]
            e0 = bv[0]
            e1 = bv[1]
            astart = pl.multiple_of((e0 // 8) * 8, 8)
            nb = (e1 - astart + K - 1) // K

            def body(b, _):
                ab = pl.multiple_of(astart + b * K, 8)
                pltpu.sync_copy(src_hbm.at[pl.ds(ab, K)], srcv)
                pltpu.sync_copy(seg_hbm.at[pl.ds(ab, K)], segv.at[pl.ds(0, K)])
                for v in range(K // 16):
                    sl = pl.ds(v * 16, 16)
                    eid = ab + v * 16 + lanes
                    m = (eid >= e0) & (eid < e1)
                    srcv[sl] = jnp.where(m, srcv[sl], 0)
                    segv[sl] = jnp.where(m, segv[sl] - base, CRT)
                pltpu.async_copy(x_hbm.at[srcv], rows, sem).wait()

                def ebody(i, _):
                    e = i * 2
                    sv = segv[pl.ds(e, 16)]
                    r0 = sv[0]
                    r1 = sv[1]
                    for c in range(DI // 16):
                        sl = pl.ds(c * 16, 16)
                        acc[r0, sl] = acc[r0, sl] + rows[e, sl]
                        acc[r1, sl] = acc[r1, sl] + rows[e + 1, sl]
                    return 0
                lax.fori_loop(0, k2_t, ebody, 0)
                return 0
            lax.fori_loop(0, nb, body, 0)

            pltpu.sync_copy(acc.at[pl.ds(0, CRT)],
                            out_hbm.at[pl.ds(base, CRT)])
            return 0
        lax.fori_loop(0, cpt_t, chunk_body, 0)

    def call(x_col, src, seg, boundsT):
        consts = jnp.array([K, DI // 16, ACR, CPT, K // 2] + [0] * 11,
                           jnp.int32)
        return agg_kernel(x_col, src, seg, boundsT, consts)
    return call


def _tc_layer(agg_cbs, cnt, x, w_cbs, root, b, relu, bn=256):
    CB = len(agg_cbs)
    _, np_, _ = agg_cbs[0].shape
    di = x.shape[1]
    do = w_cbs[0].shape[2]
    grid = (np_ // bn,)

    def body(*refs):
        agg_refs = refs[:CB]
        cnt_ref, x_ref = refs[CB], refs[CB + 1]
        w_refs = refs[CB + 2:2 * CB + 2]
        root_ref, b_ref, o_ref = refs[2 * CB + 2:]
        acc = jnp.dot(x_ref[...], root_ref[...],
                      preferred_element_type=jnp.float32)
        acc += b_ref[...]
        for r in range(R):
            inv = 1.0 / jnp.maximum(cnt_ref[:, r:r + 1], 1.0)
            pr = jnp.zeros_like(acc)
            for cb in range(CB):
                pr += jnp.dot(agg_refs[cb][r], w_refs[cb][r],
                              preferred_element_type=jnp.float32)
            acc += pr * inv
        if relu:
            acc = jnp.maximum(acc, 0.0)
        o_ref[...] = acc

    in_specs = (
        [pl.BlockSpec((R, bn, DI), lambda i: (0, i, 0)) for _ in range(CB)]
        + [pl.BlockSpec((bn, R), lambda i: (i, 0)),
           pl.BlockSpec((bn, di), lambda i: (i, 0))]
        + [pl.BlockSpec((R, DI, do), lambda i: (0, 0, 0)) for _ in range(CB)]
        + [pl.BlockSpec((di, do), lambda i: (0, 0)),
           pl.BlockSpec((1, do), lambda i: (0, 0))]
    )
    return pl.pallas_call(
        body,
        grid=grid,
        in_specs=in_specs,
        out_specs=pl.BlockSpec((bn, do), lambda i: (i, 0)),
        out_shape=jax.ShapeDtypeStruct((np_, do), jnp.float32),
    )(*agg_cbs, cnt, x, *w_cbs, root, b)


def _tc_pool(batch16, h, bn=512):
    np_, do = h.shape
    grid = (np_ // bn,)
    last = np_ // bn - 1

    def body(batch_ref, h_ref, o_ref, sum_ref, cnt_ref):
        i = pl.program_id(0)

        @pl.when(i == 0)
        def _():
            sum_ref[...] = jnp.zeros_like(sum_ref)
            cnt_ref[...] = jnp.zeros_like(cnt_ref)

        gids = lax.broadcasted_iota(jnp.int32, (bn, G), 1)
        onehot = (batch_ref[...] == gids).astype(jnp.float32)
        sum_ref[...] += lax.dot_general(
            onehot, h_ref[...], (((0,), (0,)), ((), ())),
            preferred_element_type=jnp.float32)
        cnt_ref[...] += lax.dot_general(
            onehot, jnp.ones((bn, do), jnp.float32), (((0,), (0,)), ((), ())),
            preferred_element_type=jnp.float32)

        @pl.when(i == last)
        def _():
            o_ref[...] = sum_ref[...] / jnp.maximum(cnt_ref[...], 1.0)

    return pl.pallas_call(
        body,
        grid=grid,
        in_specs=[
            pl.BlockSpec((bn, G), lambda i: (i, 0)),
            pl.BlockSpec((bn, do), lambda i: (i, 0)),
        ],
        out_specs=pl.BlockSpec((G, do), lambda i: (0, 0)),
        out_shape=jax.ShapeDtypeStruct((G, do), jnp.float32),
        scratch_shapes=[
            pltpu.VMEM((G, do), jnp.float32),
            pltpu.VMEM((G, do), jnp.float32),
        ],
    )(batch16, h)


def _sc_layer_agg(sc, h, src_p, seg_p, boundsT):
    CB = h.shape[1] // DI
    cols = jnp.split(h, CB, axis=1)
    return [sc(c, src_p, seg_p, boundsT).reshape(R, NP_, DI) for c in cols]


def kernel(x, edge_index, edge_attr, batch, w1, root1, b1, w2, root2, b2,
           w3, root3, b3, w4, root4, b4):
    x = x.astype(jnp.float32)
    src = edge_index[0].astype(jnp.int32)
    dst = edge_index[1].astype(jnp.int32)
    et = edge_attr.astype(jnp.int32)

    seg = et * NP_ + dst
    seg_s, src_s = lax.sort_key_val(seg, src)
    src_p = jnp.pad(src_s, (0, EP - E))
    seg_p = jnp.pad(seg_s, (0, EP - E), constant_values=NSEGP)

    b_ = jnp.searchsorted(seg_s, jnp.arange(NCH + 1) * CRT)
    b_ = jnp.pad(b_, (0, 16), constant_values=E).astype(jnp.int32)
    win = jnp.arange(NCH)[:, None] + jnp.arange(16)[None, :]
    boundsT = b_[win].reshape(-1)

    x_p = jnp.pad(x, ((0, NP_ - N), (0, 0)))
    batch_p = jnp.pad(batch.astype(jnp.int32), (0, NP_ - N),
                      constant_values=G)
    batch16 = jnp.broadcast_to(batch_p[:, None], (NP_, G))

    OUTP = 128
    w4p = jnp.pad(w4, ((0, 0), (0, 0), (0, OUTP - w4.shape[2])))
    root4p = jnp.pad(root4, ((0, 0), (0, OUTP - root4.shape[1])))
    b4p = jnp.pad(b4, (0, OUTP - b4.shape[0]))

    sc = _make_sc_agg()

    def wsplit(w):
        return [w[:, cb * DI:(cb + 1) * DI, :] for cb in range(w.shape[1] // DI)]

    ones_tab = jnp.ones((NP_, DI), jnp.float32)
    cnt_raw = sc(ones_tab, src_p, seg_p, boundsT)
    cnt = cnt_raw[:, 0].reshape(R, NP_).T

    a1 = _sc_layer_agg(sc, x_p, src_p, seg_p, boundsT)
    h = _tc_layer(a1, cnt, x_p, wsplit(w1), root1, b1.reshape(1, -1),
                  relu=True)

    a2 = _sc_layer_agg(sc, h, src_p, seg_p, boundsT)
    h = _tc_layer(a2, cnt, h, wsplit(w2), root2, b2.reshape(1, -1),
                  relu=True)

    a3 = _sc_layer_agg(sc, h, src_p, seg_p, boundsT)
    h = _tc_layer(a3, cnt, h, wsplit(w3), root3, b3.reshape(1, -1),
                  relu=True)

    a4 = _sc_layer_agg(sc, h, src_p, seg_p, boundsT)
    h = _tc_layer(a4, cnt, h, wsplit(w4p), root4p, b4p.reshape(1, -1),
                  relu=False)

    pooled = _tc_pool(batch16, h)
    return pooled[:, :19]
